# Initial kernel scaffold; baseline (speedup 1.0000x reference)
import functools

import jax
import jax.numpy as jnp
from jax import lax
from jax.experimental import pallas as pl
from jax.experimental.pallas import tpu as pltpu
from jax.experimental.pallas import tpu_sc as plsc

N_NODES = 10000
N_EDGES = 320000
D = 128

NC = 2
NS = 16
NW = NC * NS

C = 80
EPW = N_EDGES // NW
NCHUNK = EPW // C
RPS = N_NODES // NS


def _sc_aggregate(row, col, val, X):
    mesh = plsc.VectorSubcoreMesh(core_axis_name="c", subcore_axis_name="s")

    @functools.partial(
        pl.kernel,
        out_type=jax.ShapeDtypeStruct((NC, N_NODES, D), jnp.float32),
        mesh=mesh,
        scratch_types=[
            pltpu.VMEM((C,), jnp.int32),
            pltpu.VMEM((C,), jnp.int32),
            pltpu.VMEM((C,), jnp.float32),
            pltpu.VMEM((C, D), jnp.float32),
            pltpu.VMEM_SHARED((N_NODES, D), jnp.float32),
            pltpu.SemaphoreType.DMA,
        ],
    )
    def agg(row_hbm, col_hbm, val_hbm, x_hbm, out_hbm,
            row_v, col_v, val_v, rows_v, acc, sem):
        c = lax.axis_index("c")
        s = lax.axis_index("s")
        wid = c * NS + s

        @pl.loop(0, C)
        def _(i):
            for j in range(0, D, 16):
                rows_v[i, pl.ds(j, 16)] = jnp.zeros((16,), jnp.float32)

        off = 0
        while off < RPS:
            n = min(C, RPS - off)
            pltpu.sync_copy(rows_v.at[pl.ds(0, n)],
                            acc.at[pl.ds(s * RPS + off, n)])
            off += n
        plsc.subcore_barrier()

        @pl.loop(0, NCHUNK)
        def _(k):
            base = wid * EPW + k * C
            pltpu.sync_copy(row_hbm.at[pl.ds(base, C)], row_v)
            pltpu.sync_copy(col_hbm.at[pl.ds(base, C)], col_v)
            pltpu.sync_copy(val_hbm.at[pl.ds(base, C)], val_v)
            pltpu.async_copy(x_hbm.at[col_v], rows_v, sem).wait()

            @pl.loop(0, C)
            def _(e):
                v = val_v[e]
                for j in range(0, D, 16):
                    rows_v[e, pl.ds(j, 16)] = rows_v[e, pl.ds(j, 16)] * v

            pltpu.sync_copy(rows_v, acc.at[row_v], add=True)

        plsc.subcore_barrier()
        pltpu.sync_copy(acc.at[pl.ds(s * RPS, RPS)],
                        out_hbm.at[c, pl.ds(s * RPS, RPS)])

    return agg(row, col, val, X)


def _tc_linear(partials, weight, bias):
    def body(p_ref, w_ref, b_ref, o_ref):
        h = p_ref[0] + p_ref[1]
        o_ref[...] = (
            jnp.dot(h, w_ref[...], preferred_element_type=jnp.float32)
            + b_ref[...]
        )

    return pl.pallas_call(
        body,
        out_shape=jax.ShapeDtypeStruct((N_NODES, D), jnp.float32),
    )(partials, weight, bias.reshape(1, D))


def kernel(edge_index, edge_values, X, weight, bias):
    row = edge_index[0]
    col = edge_index[1]
    partials = _sc_aggregate(row, col, edge_values, X)
    return _tc_linear(partials, weight, bias)


# speedup vs baseline: 4.5071x; 4.5071x over previous
import functools

import jax
import jax.numpy as jnp
from jax import lax
from jax.experimental import pallas as pl
from jax.experimental.pallas import tpu as pltpu
from jax.experimental.pallas import tpu_sc as plsc

N_NODES = 10000
N_EDGES = 320000
D = 128

NC = 2
NS = 16
NW = NC * NS

C = 80
EPW = N_EDGES // NW
NCHUNK = EPW // C
N_PAD = 10240
RPS = N_PAD // NS


def _sc_aggregate(row, col, val, X):
    mesh = plsc.VectorSubcoreMesh(core_axis_name="c", subcore_axis_name="s")

    @functools.partial(
        pl.kernel,
        out_type=jax.ShapeDtypeStruct((NC, N_PAD, D), jnp.float32),
        mesh=mesh,
        scratch_types=[
            pltpu.VMEM((C,), jnp.int32),
            pltpu.VMEM((C,), jnp.int32),
            pltpu.VMEM((C,), jnp.float32),
            pltpu.VMEM((C, D), jnp.float32),
            pltpu.VMEM_SHARED((N_PAD, D), jnp.float32),
            pltpu.SemaphoreType.DMA,
        ],
    )
    def agg(row_hbm, col_hbm, val_hbm, x_hbm, out_hbm,
            row_v, col_v, val_v, rows_v, acc, sem):
        c = lax.axis_index("c")
        s = lax.axis_index("s")
        wid = c * NS + s

        @pl.loop(0, C)
        def _(i):
            for j in range(0, D, 16):
                rows_v[i, pl.ds(j, 16)] = jnp.zeros((16,), jnp.float32)

        off = 0
        while off < RPS:
            n = min(C, RPS - off)
            pltpu.sync_copy(rows_v.at[pl.ds(0, n)],
                            acc.at[pl.ds(s * RPS + off, n)])
            off += n
        plsc.subcore_barrier()

        @pl.loop(0, NCHUNK)
        def _(k):
            base = wid * EPW + k * C
            pltpu.sync_copy(row_hbm.at[pl.ds(base, C)], row_v)
            pltpu.sync_copy(col_hbm.at[pl.ds(base, C)], col_v)
            pltpu.sync_copy(val_hbm.at[pl.ds(base, C)], val_v)
            pltpu.async_copy(x_hbm.at[col_v], rows_v, sem).wait()

            @pl.loop(0, C, step=16)
            def _(g):
                val16 = val_v[pl.ds(g, 16)]
                for i in range(16):
                    v = val16[i]
                    for j in range(0, D, 16):
                        rows_v[g + i, pl.ds(j, 16)] = (
                            rows_v[g + i, pl.ds(j, 16)] * v)

            pltpu.sync_copy(rows_v, acc.at[row_v], add=True)

        plsc.subcore_barrier()
        pltpu.sync_copy(acc.at[pl.ds(s * RPS, RPS)],
                        out_hbm.at[c, pl.ds(s * RPS, RPS)])

    return agg(row, col, val, X)


def _tc_linear(partials, weight, bias):
    def body(p_ref, w_ref, b_ref, o_ref):
        h = p_ref[0] + p_ref[1]
        o_ref[...] = (
            jnp.dot(h, w_ref[...], preferred_element_type=jnp.float32)
            + b_ref[...]
        )

    return pl.pallas_call(
        body,
        out_shape=jax.ShapeDtypeStruct((N_NODES, D), jnp.float32),
    )(partials, weight, bias.reshape(1, D))


def kernel(edge_index, edge_values, X, weight, bias):
    row = edge_index[0]
    col = edge_index[1]
    partials = _sc_aggregate(row, col, edge_values, X)[:, :N_NODES, :]
    return _tc_linear(partials, weight, bias)


# speedup vs baseline: 8.2415x; 1.8286x over previous
import functools

import jax
import jax.numpy as jnp
from jax import lax
from jax.experimental import pallas as pl
from jax.experimental.pallas import tpu as pltpu
from jax.experimental.pallas import tpu_sc as plsc

N_NODES = 10000
N_EDGES = 320000
D = 128

NC = 2
NS = 16
NW = NC * NS

C = 80
EPW = N_EDGES // NW
NCHUNK = EPW // C
N_PAD = 10240
RPS = N_PAD // NS


def _sc_aggregate(row, col, val, X):
    mesh = plsc.VectorSubcoreMesh(core_axis_name="c", subcore_axis_name="s")

    @functools.partial(
        pl.kernel,
        out_type=jax.ShapeDtypeStruct((NC, N_PAD, D), jnp.float32),
        mesh=mesh,
        scratch_types=(
            [pltpu.VMEM((C,), jnp.int32)] * 3
            + [pltpu.VMEM((C,), jnp.int32)] * 3
            + [pltpu.VMEM((C,), jnp.float32)] * 3
            + [pltpu.VMEM((C, D), jnp.float32)] * 3
            + [pltpu.VMEM_SHARED((N_PAD, D), jnp.float32)]
            + [pltpu.SemaphoreType.DMA] * 9
        ),
    )
    def agg(row_hbm, col_hbm, val_hbm, x_hbm, out_hbm,
            row0, row1, row2, col0, col1, col2, val0, val1, val2,
            rows0, rows1, rows2, acc,
            si0, si1, si2, sg0, sg1, sg2, ss0, ss1, ss2):
        cc = lax.axis_index("c")
        s = lax.axis_index("s")
        wid = cc * NS + s

        row_b = (row0, row1, row2)
        col_b = (col0, col1, col2)
        val_b = (val0, val1, val2)
        rows_b = (rows0, rows1, rows2)
        si = (si0, si1, si2)
        sg = (sg0, sg1, sg2)
        ss = (ss0, ss1, ss2)

        def idx_start(chunk, b):
            base = wid * EPW + chunk * C
            pltpu.async_copy(row_hbm.at[pl.ds(base, C)], row_b[b], si[b])
            pltpu.async_copy(col_hbm.at[pl.ds(base, C)], col_b[b], si[b])
            pltpu.async_copy(val_hbm.at[pl.ds(base, C)], val_b[b], si[b])

        def idx_wait(b):
            pltpu.make_async_copy(
                row_hbm.at[pl.ds(0, C)], row_b[b], si[b]).wait()
            pltpu.make_async_copy(
                col_hbm.at[pl.ds(0, C)], col_b[b], si[b]).wait()
            pltpu.make_async_copy(
                val_hbm.at[pl.ds(0, C)], val_b[b], si[b]).wait()

        def gather_start(b):
            pltpu.async_copy(x_hbm.at[col_b[b]], rows_b[b], sg[b])

        def gather_wait(b):
            pltpu.make_async_copy(x_hbm.at[col_b[b]], rows_b[b], sg[b]).wait()

        def scatter_start(b):
            pltpu.make_async_copy(
                rows_b[b], acc.at[row_b[b]], ss[b]).start(add=True)

        def scatter_wait(b):
            pltpu.make_async_copy(rows_b[b], acc.at[row_b[b]], ss[b]).wait()

        def scale(b):
            rv = rows_b[b]
            vv = val_b[b]

            @pl.loop(0, C, step=16)
            def _(g):
                val16 = vv[pl.ds(g, 16)]
                for i in range(16):
                    v = val16[i]
                    for j in range(0, D, 16):
                        rv[g + i, pl.ds(j, 16)] = rv[g + i, pl.ds(j, 16)] * v

        @pl.loop(0, C)
        def _(i):
            for j in range(0, D, 16):
                rows0[i, pl.ds(j, 16)] = jnp.zeros((16,), jnp.float32)

        off = 0
        while off < RPS:
            n = min(C, RPS - off)
            pltpu.sync_copy(rows0.at[pl.ds(0, n)],
                            acc.at[pl.ds(s * RPS + off, n)])
            off += n

        idx_start(0, 0)
        idx_wait(0)
        idx_start(1, 1)
        gather_start(0)
        plsc.subcore_barrier()

        gather_wait(0)
        scale(0)
        scatter_start(0)
        idx_start(2, 2)
        idx_wait(1)
        gather_start(1)

        @pl.loop(0, (NCHUNK - 2) // 3)
        def _(k):
            c0 = 1 + k * 3
            for j in range(3):
                b = (1 + j) % 3
                b1 = (2 + j) % 3
                b2 = (3 + j) % 3
                c = c0 + j
                gather_wait(b)
                scale(b)
                scatter_start(b)
                scatter_wait(b2)

                @pl.when(c + 2 < NCHUNK)
                def _():
                    idx_start(c + 2, b2)

                idx_wait(b1)
                gather_start(b1)

        gather_wait(1)
        scale(1)
        scatter_start(1)
        scatter_wait(0)
        scatter_wait(1)

        plsc.subcore_barrier()
        pltpu.sync_copy(acc.at[pl.ds(s * RPS, RPS)],
                        out_hbm.at[cc, pl.ds(s * RPS, RPS)])

    return agg(row, col, val, X)


def _tc_linear(partials, weight, bias):
    def body(p_ref, w_ref, b_ref, o_ref):
        h = p_ref[0] + p_ref[1]
        o_ref[...] = (
            jnp.dot(h, w_ref[...], preferred_element_type=jnp.float32)
            + b_ref[...]
        )

    return pl.pallas_call(
        body,
        out_shape=jax.ShapeDtypeStruct((N_NODES, D), jnp.float32),
    )(partials, weight, bias.reshape(1, D))


def kernel(edge_index, edge_values, X, weight, bias):
    row = edge_index[0]
    col = edge_index[1]
    partials = _sc_aggregate(row, col, edge_values, X)[:, :N_NODES, :]
    return _tc_linear(partials, weight, bias)
